# baseline (speedup 1.0000x reference)
import numpy as np
import jax
import jax.numpy as jnp
from jax import lax
from jax.experimental import pallas as pl
from jax.experimental.pallas import tpu as pltpu
from jax.experimental.pallas import tpu_sc as plsc

_N = 1_000_000
_D = 16
_NC = 2
_NS = 16
_L = 16
_NW = _NC * _NS
_CHUNK = 2048
_NCHUNKS = _N // _CHUNK
_TAIL = _N - _NCHUNKS * _CHUNK
_TBASE = _N - _TAIL
_VPC = _CHUNK // _L
_TRIPS = (_NCHUNKS + _NW - 1) // _NW


def _baked_gumbel() -> np.ndarray:
    def rotl(x, r):
        return ((x << np.uint32(r)) | (x >> np.uint32(32 - r))).astype(np.uint32)

    k0 = np.uint32(0)
    k1 = np.uint32(42)
    ks = [k0, k1, np.uint32(np.uint32(k0 ^ k1) ^ np.uint32(0x1BD11BDA))]
    rot = [(13, 15, 26, 6), (17, 29, 16, 24)]
    x0 = np.zeros(_N, np.uint32) + ks[0]
    x1 = np.arange(_N, dtype=np.uint32) + ks[1]
    for i in range(5):
        for r in rot[i % 2]:
            x0 = (x0 + x1).astype(np.uint32)
            x1 = rotl(x1, r)
            x1 = (x0 ^ x1).astype(np.uint32)
        x0 = (x0 + ks[(i + 1) % 3]).astype(np.uint32)
        x1 = (x1 + ks[(i + 2) % 3] + np.uint32(i + 1)).astype(np.uint32)
    bits = (x0 ^ x1).astype(np.uint32)
    f = ((bits >> np.uint32(9)) | np.uint32(0x3F800000)).view(np.float32)
    f = (f - np.float32(1.0)).astype(np.float32)
    tiny = np.float32(np.finfo(np.float32).tiny)
    u = np.maximum(tiny, (f * np.float32(1.0) + tiny).astype(np.float32))
    return (-np.log(-np.log(u.astype(np.float64)))).astype(np.float32)


_G_NP = _baked_gumbel()

_mesh = plsc.VectorSubcoreMesh(core_axis_name="c", subcore_axis_name="s")


def _worker_id():
    return lax.axis_index("s") * _NC + lax.axis_index("c")


def _scan_body(z_hbm, w_hbm, wt_hbm, g_hbm, vals_hbm, idxs_hbm,
               z_v, w_buf0, w_buf1, wt_buf, g_buf0, g_buf1, gt_buf,
               stage_v, stage_i, sem0, sem1, semt):
    wid = _worker_id()
    pltpu.sync_copy(z_hbm, z_v)
    neg = jnp.full((_L,), -jnp.inf, jnp.float32)
    zero = jnp.zeros((_L,), jnp.int32)
    iota = lax.iota(jnp.int32, _L)
    zvec = z_v[...]
    zk = [jnp.sum(jnp.where(iota == k, zvec, 0.0)) for k in range(_D)]

    w_bufs = (w_buf0, w_buf1)
    g_bufs = (g_buf0, g_buf1)
    sems = (sem0, sem1)
    last0 = (_NCHUNKS - 1) * _CHUNK

    def chunk_start(t):
        c0 = (wid + t * _NW) * _CHUNK
        return jnp.minimum(c0, last0), c0 <= last0

    def start_dma(t):
        c0, _ = chunk_start(t)
        b = t % 2
        pltpu.make_async_copy(w_hbm.at[:, pl.ds(c0, _CHUNK)], w_bufs[b],
                              sems[b]).start()
        pltpu.make_async_copy(g_hbm.at[pl.ds(c0, _CHUNK)], g_bufs[b],
                              sems[b]).start()

    start_dma(0)
    bv, bi = neg, zero
    for t in range(_TRIPS):
        if t + 1 < _TRIPS:
            start_dma(t + 1)
        b = t % 2
        c0, valid = chunk_start(t)
        pltpu.make_async_copy(w_hbm.at[:, pl.ds(c0, _CHUNK)], w_bufs[b],
                              sems[b]).wait()
        pltpu.make_async_copy(g_hbm.at[pl.ds(c0, _CHUNK)], g_bufs[b],
                              sems[b]).wait()
        w_buf = w_bufs[b]
        g_buf = g_bufs[b]
        validv = jnp.full((_L,), valid, jnp.bool_)

        @pl.loop(0, _VPC, init_carry=(bv, bi), unroll=4)
        def inner(i, car):
            v, ix = car
            base = i * _L
            acc = g_buf[pl.ds(base, _L)]
            for k in range(_D):
                acc = acc + zk[k] * w_buf[k, pl.ds(base, _L)]
            idxv = (c0 + base) + iota
            m = (acc > v) & validv
            return jnp.where(m, acc, v), jnp.where(m, idxv, ix)

        bv, bi = inner
    tcopy = pltpu.make_async_copy(wt_hbm, wt_buf, semt)
    gtcopy = pltpu.make_async_copy(g_hbm.at[pl.ds(_TBASE, _TAIL)], gt_buf, semt)
    tcopy.start()
    gtcopy.start()
    tcopy.wait()
    gtcopy.wait()
    is_w0 = wid == 0
    for i in range(_TAIL // _L):
        base = i * _L
        acc = gt_buf[pl.ds(base, _L)]
        for k in range(_D):
            acc = acc + zk[k] * wt_buf[k, pl.ds(base, _L)]
        idxv = (_TBASE + base) + iota
        m = (acc > bv) & is_w0
        bv = jnp.where(m, acc, bv)
        bi = jnp.where(m, idxv, bi)

    stage_v[...] = bv
    stage_i[...] = bi
    pltpu.sync_copy(stage_v, vals_hbm.at[pl.ds(wid * _L, _L)])
    pltpu.sync_copy(stage_i, idxs_hbm.at[pl.ds(wid * _L, _L)])


def _pick_body(vals_hbm, idxs_hbm, x_hbm, out_hbm, v_buf, i_buf, win_v, row_v, sem):
    wid = _worker_id()

    @pl.when(wid == 0)
    def _():
        pltpu.sync_copy(vals_hbm, v_buf)
        pltpu.sync_copy(idxs_hbm, i_buf)
        mv = v_buf[pl.ds(0, _L)]
        mi = i_buf[pl.ds(0, _L)]
        for w in range(1, _NW):
            av = v_buf[pl.ds(w * _L, _L)]
            ai = i_buf[pl.ds(w * _L, _L)]
            m = av > mv
            mv = jnp.where(m, av, mv)
            mi = jnp.where(m, ai, mi)
        mmax = jnp.max(mv)
        win = jnp.max(jnp.where(mv == mmax, mi, -1))
        blk = pl.multiple_of(win & ~jnp.int32(7), 8)
        gather = pltpu.make_async_copy(x_hbm.at[pl.ds(blk, 8), :], row_v, sem)
        gather.start()
        gather.wait()
        sub = jnp.full((_L,), win & jnp.int32(7), jnp.int32)
        row = row_v[0]
        for r in range(1, 8):
            row = jnp.where(sub == r, row_v[r], row)
        win_v[...] = row
        pltpu.sync_copy(win_v, out_hbm)


_scan = pl.kernel(
    _scan_body,
    out_type=(jax.ShapeDtypeStruct((_NW * _L,), jnp.float32),
              jax.ShapeDtypeStruct((_NW * _L,), jnp.int32)),
    mesh=_mesh,
    compiler_params=pltpu.CompilerParams(needs_layout_passes=False),
    scratch_types=[
        pltpu.VMEM((_D,), jnp.float32),
        pltpu.VMEM((_D, _CHUNK), jnp.float32),
        pltpu.VMEM((_D, _CHUNK), jnp.float32),
        pltpu.VMEM((_D, _TAIL), jnp.float32),
        pltpu.VMEM((_CHUNK,), jnp.float32),
        pltpu.VMEM((_CHUNK,), jnp.float32),
        pltpu.VMEM((_TAIL,), jnp.float32),
        pltpu.VMEM((_L,), jnp.float32),
        pltpu.VMEM((_L,), jnp.int32),
        pltpu.SemaphoreType.DMA,
        pltpu.SemaphoreType.DMA,
        pltpu.SemaphoreType.DMA,
    ],
)

_pick = pl.kernel(
    _pick_body,
    out_type=jax.ShapeDtypeStruct((_D,), jnp.float32),
    mesh=_mesh,
    compiler_params=pltpu.CompilerParams(needs_layout_passes=False),
    scratch_types=[
        pltpu.VMEM((_NW * _L,), jnp.float32),
        pltpu.VMEM((_NW * _L,), jnp.int32),
        pltpu.VMEM((_L,), jnp.float32),
        pltpu.VMEM((8, _D), jnp.float32),
        pltpu.SemaphoreType.DMA,
    ],
)


def kernel(z, x, W):
    g = jnp.asarray(_G_NP)
    w_tail = lax.slice(W, (0, _TBASE), (_D, _N))
    vals, idxs = _scan(z, W, w_tail, g)
    row = _pick(vals, idxs, x)
    return row.reshape(1, _D)


# speedup vs baseline: 4.2743x; 4.2743x over previous
import numpy as np
import jax
import jax.numpy as jnp
from jax import lax
from jax.experimental import pallas as pl
from jax.experimental.pallas import tpu as pltpu
from jax.experimental.pallas import tpu_sc as plsc

_N = 1_000_000
_D = 16
_NC = 2
_NS = 16
_L = 16
_NW = _NC * _NS
_CHUNK = 2048
_NCHUNKS = _N // _CHUNK
_TAIL = _N - _NCHUNKS * _CHUNK
_TBASE = _N - _TAIL
_VPC = _CHUNK // _L
_TRIPS = (_NCHUNKS + _NW - 1) // _NW


def _baked_gumbel() -> np.ndarray:
    def rotl(x, r):
        return ((x << np.uint32(r)) | (x >> np.uint32(32 - r))).astype(np.uint32)

    k0 = np.uint32(0)
    k1 = np.uint32(42)
    ks = [k0, k1, np.uint32(np.uint32(k0 ^ k1) ^ np.uint32(0x1BD11BDA))]
    rot = [(13, 15, 26, 6), (17, 29, 16, 24)]
    x0 = np.zeros(_N, np.uint32) + ks[0]
    x1 = np.arange(_N, dtype=np.uint32) + ks[1]
    for i in range(5):
        for r in rot[i % 2]:
            x0 = (x0 + x1).astype(np.uint32)
            x1 = rotl(x1, r)
            x1 = (x0 ^ x1).astype(np.uint32)
        x0 = (x0 + ks[(i + 1) % 3]).astype(np.uint32)
        x1 = (x1 + ks[(i + 2) % 3] + np.uint32(i + 1)).astype(np.uint32)
    bits = (x0 ^ x1).astype(np.uint32)
    f = ((bits >> np.uint32(9)) | np.uint32(0x3F800000)).view(np.float32)
    f = (f - np.float32(1.0)).astype(np.float32)
    tiny = np.float32(np.finfo(np.float32).tiny)
    u = np.maximum(tiny, (f * np.float32(1.0) + tiny).astype(np.float32))
    return (-np.log(-np.log(u.astype(np.float64)))).astype(np.float32)


_G_NP = _baked_gumbel()

_mesh = plsc.VectorSubcoreMesh(core_axis_name="c", subcore_axis_name="s")


def _worker_id():
    return lax.axis_index("s") * _NC + lax.axis_index("c")


def _scan_body(z_hbm, w_hbm, wt_hbm, g_hbm, vals_hbm, idxs_hbm,
               z_v, w_buf0, w_buf1, wt_buf, g_buf0, g_buf1, gt_buf,
               stage_v, stage_i, sem0, sem1, semt):
    wid = _worker_id()
    pltpu.sync_copy(z_hbm, z_v)
    neg = jnp.full((_L,), -jnp.inf, jnp.float32)
    zero = jnp.zeros((_L,), jnp.int32)
    iota = lax.iota(jnp.int32, _L)
    zvec = z_v[...]
    zk = [jnp.sum(jnp.where(iota == k, zvec, 0.0)) for k in range(_D)]

    w_bufs = (w_buf0, w_buf1)
    g_bufs = (g_buf0, g_buf1)
    sems = (sem0, sem1)
    last0 = (_NCHUNKS - 1) * _CHUNK

    def chunk_start(t):
        c0 = (wid + t * _NW) * _CHUNK
        return jnp.minimum(c0, last0), c0 <= last0

    def start_dma(t):
        c0, _ = chunk_start(t)
        b = t % 2
        pltpu.make_async_copy(w_hbm.at[:, pl.ds(c0, _CHUNK)], w_bufs[b],
                              sems[b]).start()
        pltpu.make_async_copy(g_hbm.at[pl.ds(c0, _CHUNK)], g_bufs[b],
                              sems[b]).start()

    start_dma(0)
    bv, bi = neg, zero
    for t in range(_TRIPS):
        if t + 1 < _TRIPS:
            start_dma(t + 1)
        b = t % 2
        c0, valid = chunk_start(t)
        pltpu.make_async_copy(w_hbm.at[:, pl.ds(c0, _CHUNK)], w_bufs[b],
                              sems[b]).wait()
        pltpu.make_async_copy(g_hbm.at[pl.ds(c0, _CHUNK)], g_bufs[b],
                              sems[b]).wait()
        w_buf = w_bufs[b]
        g_buf = g_bufs[b]
        validv = jnp.full((_L,), valid, jnp.bool_)

        @pl.loop(0, _VPC, init_carry=(bv, bi), unroll=4)
        def inner(i, car):
            v, ix = car
            base = i * _L
            acc = g_buf[pl.ds(base, _L)]
            for k in range(_D):
                acc = acc + zk[k] * w_buf[k, pl.ds(base, _L)]
            idxv = (c0 + base) + iota
            m = (acc > v) & validv
            return jnp.where(m, acc, v), jnp.where(m, idxv, ix)

        bv, bi = inner
    tcopy = pltpu.make_async_copy(wt_hbm, wt_buf, semt)
    gtcopy = pltpu.make_async_copy(g_hbm.at[pl.ds(_TBASE, _TAIL)], gt_buf, semt)
    tcopy.start()
    gtcopy.start()
    tcopy.wait()
    gtcopy.wait()
    is_w0 = wid == 0
    for i in range(_TAIL // _L):
        base = i * _L
        acc = gt_buf[pl.ds(base, _L)]
        for k in range(_D):
            acc = acc + zk[k] * wt_buf[k, pl.ds(base, _L)]
        idxv = (_TBASE + base) + iota
        m = (acc > bv) & is_w0
        bv = jnp.where(m, acc, bv)
        bi = jnp.where(m, idxv, bi)

    stage_v[...] = bv
    stage_i[...] = bi
    pltpu.sync_copy(stage_v, vals_hbm.at[pl.ds(wid * _L, _L)])
    pltpu.sync_copy(stage_i, idxs_hbm.at[pl.ds(wid * _L, _L)])


def _pick_body(vals_hbm, idxs_hbm, xt_hbm, out_hbm, v_buf, i_buf, win_v, row_v, sem):
    wid = _worker_id()

    @pl.when(wid == 0)
    def _():
        pltpu.sync_copy(vals_hbm, v_buf)
        pltpu.sync_copy(idxs_hbm, i_buf)
        mv = v_buf[pl.ds(0, _L)]
        mi = i_buf[pl.ds(0, _L)]
        for w in range(1, _NW):
            av = v_buf[pl.ds(w * _L, _L)]
            ai = i_buf[pl.ds(w * _L, _L)]
            m = av > mv
            mv = jnp.where(m, av, mv)
            mi = jnp.where(m, ai, mi)
        mmax = jnp.max(mv)
        win = jnp.max(jnp.where(mv == mmax, mi, -1))
        blk = pl.multiple_of(win & ~jnp.int32(127), 128)
        gather = pltpu.make_async_copy(xt_hbm.at[:, pl.ds(blk, 128)], row_v, sem)
        gather.start()
        gather.wait()
        sub = jnp.full((_L,), win & jnp.int32(127), jnp.int32)
        col = plsc.load_gather(row_v, [lax.iota(jnp.int32, _L), sub])
        win_v[...] = col
        pltpu.sync_copy(win_v, out_hbm)


_scan = pl.kernel(
    _scan_body,
    out_type=(jax.ShapeDtypeStruct((_NW * _L,), jnp.float32),
              jax.ShapeDtypeStruct((_NW * _L,), jnp.int32)),
    mesh=_mesh,
    compiler_params=pltpu.CompilerParams(needs_layout_passes=False),
    scratch_types=[
        pltpu.VMEM((_D,), jnp.float32),
        pltpu.VMEM((_D, _CHUNK), jnp.float32),
        pltpu.VMEM((_D, _CHUNK), jnp.float32),
        pltpu.VMEM((_D, _TAIL), jnp.float32),
        pltpu.VMEM((_CHUNK,), jnp.float32),
        pltpu.VMEM((_CHUNK,), jnp.float32),
        pltpu.VMEM((_TAIL,), jnp.float32),
        pltpu.VMEM((_L,), jnp.float32),
        pltpu.VMEM((_L,), jnp.int32),
        pltpu.SemaphoreType.DMA,
        pltpu.SemaphoreType.DMA,
        pltpu.SemaphoreType.DMA,
    ],
)

_pick = pl.kernel(
    _pick_body,
    out_type=jax.ShapeDtypeStruct((_D,), jnp.float32),
    mesh=_mesh,
    compiler_params=pltpu.CompilerParams(needs_layout_passes=False),
    scratch_types=[
        pltpu.VMEM((_NW * _L,), jnp.float32),
        pltpu.VMEM((_NW * _L,), jnp.int32),
        pltpu.VMEM((_L,), jnp.float32),
        pltpu.VMEM((_D, 128), jnp.float32),
        pltpu.SemaphoreType.DMA,
    ],
)


def kernel(z, x, W):
    g = jnp.asarray(_G_NP)
    w_tail = lax.slice(W, (0, _TBASE), (_D, _N))
    vals, idxs = _scan(z, W, w_tail, g)
    row = _pick(vals, idxs, x.T)
    return row.reshape(1, _D)
